# baseline (speedup 1.0000x reference)
import functools

import jax
import jax.numpy as jnp
from jax import lax
from jax.experimental import pallas as pl
from jax.experimental.pallas import tpu as pltpu
from jax.experimental.pallas import tpu_sc as plsc


def _mm_silu_body(x_ref, w_ref, b_ref, o_ref):
    y = jnp.dot(x_ref[...], w_ref[...].T, preferred_element_type=jnp.float32)
    y = y + b_ref[...]
    o_ref[...] = y * jax.nn.sigmoid(y)


def _class_table(emb_pad, W, b2):
    Vp, E = emb_pad.shape
    O = W.shape[0]
    NB = 512
    grid = (O // NB,)
    return pl.pallas_call(
        _mm_silu_body,
        grid=grid,
        in_specs=[
            pl.BlockSpec((Vp, E), lambda j: (0, 0)),
            pl.BlockSpec((NB, E), lambda j: (j, 0)),
            pl.BlockSpec((1, NB), lambda j: (0, j)),
        ],
        out_specs=pl.BlockSpec((Vp, NB), lambda j: (0, j)),
        out_shape=jax.ShapeDtypeStruct((Vp, O), jnp.float32),
    )(emb_pad, W, b2)


_NC, _NS = 2, 16
_NW = _NC * _NS
_CB = 8


def _gather_rows(P, idx3, B, O, nchunks):
    bpw = B // _NW
    mesh = plsc.VectorSubcoreMesh(core_axis_name="c", subcore_axis_name="s")

    @functools.partial(
        pl.kernel,
        mesh=mesh,
        out_type=jax.ShapeDtypeStruct((B, O), jnp.float32),
        scratch_types=[
            pltpu.VMEM((nchunks, _CB), jnp.int32),
            pltpu.VMEM((_CB, O), jnp.float32),
            pltpu.VMEM((_CB, O), jnp.float32),
            pltpu.SemaphoreType.DMA,
            pltpu.SemaphoreType.DMA,
        ],
    )
    def k(p_hbm, idx_hbm, out_hbm, idx_v, buf0, buf1, sem0, sem1):
        wid = lax.axis_index("s") * _NC + lax.axis_index("c")
        base = wid * bpw
        pltpu.sync_copy(idx_hbm.at[wid], idx_v)
        bufs = (buf0, buf1)
        sems = (sem0, sem1)
        pltpu.async_copy(p_hbm.at[idx_v.at[0]], bufs[0], sems[0])
        for c in range(nchunks):
            buf, sem = bufs[c % 2], sems[c % 2]
            pltpu.make_async_copy(p_hbm.at[idx_v.at[c]], buf, sem).wait()
            if c + 1 < nchunks:
                nb = (c + 1) % 2
                pltpu.async_copy(p_hbm.at[idx_v.at[c + 1]], bufs[nb], sems[nb])
            pltpu.sync_copy(buf, out_hbm.at[pl.ds(base + c * _CB, _CB)])

    return k(P, idx3)


def kernel(label, emb_table, W, b):
    B = label.shape[0]
    V, E = emb_table.shape
    O = W.shape[0]
    Vp = (V + 255) // 256 * 256
    emb_pad = jnp.pad(emb_table, ((0, Vp - V), (0, 0)))
    P = _class_table(emb_pad, W, jnp.reshape(b, (1, O)))
    nchunks = B // (_NW * _CB)
    idx3 = jnp.reshape(label.astype(jnp.int32), (_NW, nchunks, _CB))
    out = _gather_rows(P, idx3, B, O, nchunks)
    return jnp.reshape(out, (B, 1, 4, 32, 32))


# speedup vs baseline: 1.0263x; 1.0263x over previous
import functools

import jax
import jax.numpy as jnp
from jax import lax
from jax.experimental import pallas as pl
from jax.experimental.pallas import tpu as pltpu
from jax.experimental.pallas import tpu_sc as plsc


def _mm_silu_body(x_ref, w_ref, b_ref, o_ref):
    y = jnp.dot(x_ref[...], w_ref[...].T, preferred_element_type=jnp.float32)
    y = y + b_ref[...]
    o_ref[...] = y * jax.nn.sigmoid(y)


def _class_table(emb_pad, W, b2):
    Vp, E = emb_pad.shape
    O = W.shape[0]
    NB = 512
    grid = (O // NB,)
    return pl.pallas_call(
        _mm_silu_body,
        grid=grid,
        in_specs=[
            pl.BlockSpec((Vp, E), lambda j: (0, 0)),
            pl.BlockSpec((NB, E), lambda j: (j, 0)),
            pl.BlockSpec((1, NB), lambda j: (0, j)),
        ],
        out_specs=pl.BlockSpec((Vp, NB), lambda j: (0, j)),
        out_shape=jax.ShapeDtypeStruct((Vp, O), jnp.float32),
    )(emb_pad, W, b2)


_NC, _NS = 2, 16
_NW = _NC * _NS
_CB = 8


def _gather_rows(P, idx3, B, O, nchunks):
    bpw = B // _NW
    mesh = plsc.VectorSubcoreMesh(core_axis_name="c", subcore_axis_name="s")

    @functools.partial(
        pl.kernel,
        mesh=mesh,
        out_type=jax.ShapeDtypeStruct((B, O), jnp.float32),
        scratch_types=[
            pltpu.VMEM((nchunks, _CB), jnp.int32),
            pltpu.VMEM((_CB, O), jnp.float32),
            pltpu.VMEM((_CB, O), jnp.float32),
            pltpu.VMEM((_CB, O), jnp.float32),
            pltpu.SemaphoreType.DMA,
            pltpu.SemaphoreType.DMA,
            pltpu.SemaphoreType.DMA,
            pltpu.SemaphoreType.DMA,
            pltpu.SemaphoreType.DMA,
            pltpu.SemaphoreType.DMA,
        ],
    )
    def k(p_hbm, idx_hbm, out_hbm, idx_v,
          buf0, buf1, buf2, g0, g1, g2, o0, o1, o2):
        wid = lax.axis_index("s") * _NC + lax.axis_index("c")
        base = wid * bpw
        pltpu.sync_copy(idx_hbm.at[wid], idx_v)
        bufs = (buf0, buf1, buf2)
        gsems = (g0, g1, g2)
        osems = (o0, o1, o2)

        def gather(c):
            pltpu.async_copy(p_hbm.at[idx_v.at[c]], bufs[c % 3], gsems[c % 3])

        def out_slice(c):
            return out_hbm.at[pl.ds(base + c * _CB, _CB)]

        gather(0)
        gather(1)
        for c in range(nchunks):
            b = c % 3
            pltpu.make_async_copy(p_hbm.at[idx_v.at[c]], bufs[b], gsems[b]).wait()
            pltpu.async_copy(bufs[b], out_slice(c), osems[b])
            if c + 2 < nchunks:
                nb = (c + 2) % 3
                if c >= 1:
                    pltpu.make_async_copy(bufs[nb], out_slice(c - 1), osems[nb]).wait()
                gather(c + 2)
        for c in range(max(0, nchunks - 3), nchunks):
            b = c % 3
            pltpu.make_async_copy(bufs[b], out_slice(c), osems[b]).wait()

    return k(P, idx3)


def kernel(label, emb_table, W, b):
    B = label.shape[0]
    V, E = emb_table.shape
    O = W.shape[0]
    Vp = (V + 255) // 256 * 256
    emb_pad = jnp.pad(emb_table, ((0, Vp - V), (0, 0)))
    P = _class_table(emb_pad, W, jnp.reshape(b, (1, O)))
    nchunks = B // (_NW * _CB)
    idx3 = jnp.reshape(label.astype(jnp.int32), (_NW, nchunks, _CB))
    out = _gather_rows(P, idx3, B, O, nchunks)
    return jnp.reshape(out, (B, 1, 4, 32, 32))


# speedup vs baseline: 2.2655x; 2.2074x over previous
import functools

import jax
import jax.numpy as jnp
from jax import lax
from jax.experimental import pallas as pl
from jax.experimental.pallas import tpu as pltpu
from jax.experimental.pallas import tpu_sc as plsc


_NC, _NS = 2, 16
_NW = _NC * _NS


def _gather_x(emb, idx2, B, E):
    bpw = B // _NW
    mesh = plsc.VectorSubcoreMesh(core_axis_name="c", subcore_axis_name="s")

    @functools.partial(
        pl.kernel,
        mesh=mesh,
        out_type=jax.ShapeDtypeStruct((B, E), jnp.float32),
        scratch_types=[
            pltpu.VMEM((bpw,), jnp.int32),
            pltpu.VMEM((bpw, E), jnp.float32),
            pltpu.SemaphoreType.DMA,
        ],
    )
    def k(emb_hbm, idx_hbm, out_hbm, idx_v, rows_v, sem):
        wid = lax.axis_index("s") * _NC + lax.axis_index("c")
        pltpu.sync_copy(idx_hbm.at[wid], idx_v)
        pltpu.async_copy(emb_hbm.at[idx_v], rows_v, sem).wait()
        pltpu.sync_copy(rows_v, out_hbm.at[pl.ds(wid * bpw, bpw)])

    return k(emb, idx2)


def _mm_body(w_ref, x_ref, b_ref, o_ref, wbf_ref):
    j = pl.program_id(0)

    @pl.when(j == 0)
    def _():
        wbf_ref[...] = w_ref[...].astype(jnp.bfloat16)

    xbf = x_ref[...].astype(jnp.bfloat16)
    y = jax.lax.dot_general(
        wbf_ref[...], xbf,
        dimension_numbers=(((1,), (1,)), ((), ())),
        preferred_element_type=jnp.float32,
    )
    y = y + b_ref[...]
    o_ref[...] = y * jax.nn.sigmoid(y)


def _mm_silu_t(W, x, b2):
    O, E = W.shape
    B = x.shape[0]
    BN = 512
    return pl.pallas_call(
        _mm_body,
        grid=(B // BN,),
        in_specs=[
            pl.BlockSpec((O, E), lambda j: (0, 0)),
            pl.BlockSpec((BN, E), lambda j: (j, 0)),
            pl.BlockSpec((O, 1), lambda j: (0, 0)),
        ],
        out_specs=pl.BlockSpec((O, BN), lambda j: (0, j)),
        out_shape=jax.ShapeDtypeStruct((O, B), jnp.float32),
        scratch_shapes=[pltpu.VMEM((O, E), jnp.bfloat16)],
    )(W, x, b2)


def kernel(label, emb_table, W, b):
    B = label.shape[0]
    E = emb_table.shape[1]
    O = W.shape[0]
    idx2 = jnp.reshape(label.astype(jnp.int32), (_NW, B // _NW))
    x = _gather_x(emb_table, idx2, B, E)
    outT = _mm_silu_t(W, x, jnp.reshape(b, (O, 1)))
    return jnp.reshape(outT.T, (B, 1, 4, 32, 32))


# speedup vs baseline: 2.3367x; 1.0314x over previous
import functools

import jax
import jax.numpy as jnp
from jax import lax
from jax.experimental import pallas as pl
from jax.experimental.pallas import tpu as pltpu
from jax.experimental.pallas import tpu_sc as plsc


_NC, _NS = 2, 16
_NW = _NC * _NS


def _gather_x(emb, label, B, E):
    bpw = B // _NW
    mesh = plsc.VectorSubcoreMesh(core_axis_name="c", subcore_axis_name="s")

    @functools.partial(
        pl.kernel,
        mesh=mesh,
        out_type=jax.ShapeDtypeStruct((B, E), jnp.float32),
        scratch_types=[
            pltpu.VMEM((bpw,), jnp.int32),
            pltpu.VMEM((bpw, E), jnp.float32),
            pltpu.SemaphoreType.DMA,
        ],
    )
    def k(emb_hbm, idx_hbm, out_hbm, idx_v, rows_v, sem):
        wid = lax.axis_index("s") * _NC + lax.axis_index("c")
        base = wid * bpw
        pltpu.sync_copy(idx_hbm.at[pl.ds(base, bpw)], idx_v)
        pltpu.async_copy(emb_hbm.at[idx_v], rows_v, sem).wait()
        pltpu.sync_copy(rows_v, out_hbm.at[pl.ds(base, bpw)])

    return k(emb, label)


def _cast_body(w_ref, o_ref):
    o_ref[...] = w_ref[...].astype(jnp.bfloat16)


def _cast_bf16(W):
    O, E = W.shape
    BO = 1024
    return pl.pallas_call(
        _cast_body,
        grid=(O // BO,),
        in_specs=[pl.BlockSpec((BO, E), lambda j: (j, 0))],
        out_specs=pl.BlockSpec((BO, E), lambda j: (j, 0)),
        out_shape=jax.ShapeDtypeStruct((O, E), jnp.bfloat16),
    )(W)


def _mm_body(w_ref, x_ref, b_ref, o_ref):
    xbf = x_ref[...].astype(jnp.bfloat16)
    y = jax.lax.dot_general(
        w_ref[...], xbf,
        dimension_numbers=(((1,), (1,)), ((), ())),
        preferred_element_type=jnp.float32,
    )
    y = y + b_ref[...]
    o_ref[...] = 0.5 * y * (1.0 + jnp.tanh(0.5 * y))


def _mm_silu_t(Wbf, x, b2):
    O, E = Wbf.shape
    B = x.shape[0]
    BO = 512
    return pl.pallas_call(
        _mm_body,
        grid=(O // BO,),
        in_specs=[
            pl.BlockSpec((BO, E), lambda j: (j, 0)),
            pl.BlockSpec((B, E), lambda j: (0, 0)),
            pl.BlockSpec((BO, 1), lambda j: (j, 0)),
        ],
        out_specs=pl.BlockSpec((BO, B), lambda j: (j, 0)),
        out_shape=jax.ShapeDtypeStruct((O, B), jnp.float32),
    )(Wbf, x, b2)


def kernel(label, emb_table, W, b):
    B = label.shape[0]
    E = emb_table.shape[1]
    O = W.shape[0]
    x = _gather_x(emb_table, label.astype(jnp.int32), B, E)
    Wbf = _cast_bf16(W)
    outT = _mm_silu_t(Wbf, x, jnp.reshape(b, (O, 1)))
    return jnp.reshape(outT.T, (B, 1, 4, 32, 32))


# speedup vs baseline: 2.3986x; 1.0265x over previous
import functools

import jax
import jax.numpy as jnp
from jax import lax
from jax.experimental import pallas as pl
from jax.experimental.pallas import tpu as pltpu
from jax.experimental.pallas import tpu_sc as plsc


_NC, _NS = 2, 16
_NW = _NC * _NS


def _gather_x(emb, label, B, E):
    bpw = B // _NW
    mesh = plsc.VectorSubcoreMesh(core_axis_name="c", subcore_axis_name="s")

    @functools.partial(
        pl.kernel,
        mesh=mesh,
        out_type=jax.ShapeDtypeStruct((B, E), jnp.float32),
        scratch_types=[
            pltpu.VMEM((bpw,), jnp.int32),
            pltpu.VMEM((bpw, E), jnp.float32),
            pltpu.SemaphoreType.DMA,
        ],
    )
    def k(emb_hbm, idx_hbm, out_hbm, idx_v, rows_v, sem):
        wid = lax.axis_index("s") * _NC + lax.axis_index("c")
        pltpu.sync_copy(idx_hbm.at[wid], idx_v)
        pltpu.async_copy(emb_hbm.at[idx_v], rows_v, sem).wait()
        pltpu.sync_copy(rows_v, out_hbm.at[pl.ds(wid * bpw, bpw)])

    return k(emb, label)


def _cast_body(w_ref, o_ref):
    o_ref[...] = w_ref[...].astype(jnp.bfloat16)


def _cast_bf16(W):
    O, E = W.shape
    BO = 1024
    return pl.pallas_call(
        _cast_body,
        grid=(O // BO,),
        in_specs=[pl.BlockSpec((BO, E), lambda j: (j, 0))],
        out_specs=pl.BlockSpec((BO, E), lambda j: (j, 0)),
        out_shape=jax.ShapeDtypeStruct((O, E), jnp.bfloat16),
    )(W)


def _mm_body(w_ref, x_ref, b_ref, o_ref, xbf_ref):
    @pl.when(pl.program_id(0) == 0)
    def _():
        xbf_ref[...] = x_ref[...].astype(jnp.bfloat16)

    y = jax.lax.dot_general(
        w_ref[...], xbf_ref[...],
        dimension_numbers=(((1,), (1,)), ((), ())),
        preferred_element_type=jnp.float32,
    )
    y = y + b_ref[...]
    o_ref[...] = 0.5 * y * (1.0 + jnp.tanh(0.5 * y))


def _mm_silu_t(Wbf, x, b2):
    O, E = Wbf.shape
    B = x.shape[0]
    BO = 512
    return pl.pallas_call(
        _mm_body,
        grid=(O // BO,),
        in_specs=[
            pl.BlockSpec((BO, E), lambda j: (j, 0)),
            pl.BlockSpec((B, E), lambda j: (0, 0)),
            pl.BlockSpec((BO, 1), lambda j: (j, 0)),
        ],
        out_specs=pl.BlockSpec((BO, B), lambda j: (j, 0)),
        out_shape=jax.ShapeDtypeStruct((O, B), jnp.float32),
        scratch_shapes=[pltpu.VMEM((B, E), jnp.bfloat16)],
    )(Wbf, x, b2)


def kernel(label, emb_table, W, b):
    B = label.shape[0]
    E = emb_table.shape[1]
    O = W.shape[0]
    idx2 = jnp.reshape(label.astype(jnp.int32), (_NW, B // _NW))
    x = _gather_x(emb_table, idx2, B, E)
    Wbf = _cast_bf16(W)
    outT = _mm_silu_t(Wbf, x, jnp.reshape(b, (O, 1)))
    return jnp.reshape(outT.T, (B, 1, 4, 32, 32))


# speedup vs baseline: 2.4003x; 1.0007x over previous
import functools

import jax
import jax.numpy as jnp
from jax import lax
from jax.experimental import pallas as pl
from jax.experimental.pallas import tpu as pltpu
from jax.experimental.pallas import tpu_sc as plsc


_NC, _NS = 2, 16
_NW = _NC * _NS


def _gather_x(emb, label, B, E):
    bpw = B // _NW
    mesh = plsc.VectorSubcoreMesh(core_axis_name="c", subcore_axis_name="s")

    @functools.partial(
        pl.kernel,
        mesh=mesh,
        out_type=jax.ShapeDtypeStruct((B, E), jnp.float32),
        scratch_types=[
            pltpu.VMEM((bpw,), jnp.int32),
            pltpu.VMEM((bpw, E), jnp.float32),
            pltpu.SemaphoreType.DMA,
        ],
    )
    def k(emb_hbm, idx_hbm, out_hbm, idx_v, rows_v, sem):
        wid = lax.axis_index("s") * _NC + lax.axis_index("c")
        pltpu.sync_copy(idx_hbm.at[wid], idx_v)
        pltpu.async_copy(emb_hbm.at[idx_v], rows_v, sem).wait()
        pltpu.sync_copy(rows_v, out_hbm.at[pl.ds(wid * bpw, bpw)])

    return k(emb, label)


def _cast_body(w_ref, o_ref):
    o_ref[...] = w_ref[...].astype(jnp.bfloat16)


def _cast_bf16(W):
    O, E = W.shape
    BO = 1024
    return pl.pallas_call(
        _cast_body,
        grid=(O // BO,),
        in_specs=[pl.BlockSpec((BO, E), lambda j: (j, 0))],
        out_specs=pl.BlockSpec((BO, E), lambda j: (j, 0)),
        out_shape=jax.ShapeDtypeStruct((O, E), jnp.bfloat16),
    )(W)


def _mm_body(w_ref, x_ref, b_ref, o_ref, xbf_ref):
    @pl.when(pl.program_id(0) == 0)
    def _():
        xbf_ref[...] = x_ref[...].astype(jnp.bfloat16)

    y = jax.lax.dot_general(
        w_ref[...], xbf_ref[...],
        dimension_numbers=(((1,), (1,)), ((), ())),
        preferred_element_type=jnp.float32,
    )
    y = y + b_ref[...]
    o_ref[...] = 0.5 * y * (1.0 + jnp.tanh(0.5 * y))


def _mm_silu_t(Wbf, x, b2):
    O, E = Wbf.shape
    B = x.shape[0]
    BO = 512
    return pl.pallas_call(
        _mm_body,
        grid=(O // BO,),
        in_specs=[
            pl.BlockSpec((BO, E), lambda j: (j, 0)),
            pl.BlockSpec((B, E), lambda j: (0, 0)),
            pl.BlockSpec((BO, 1), lambda j: (j, 0)),
        ],
        out_specs=pl.BlockSpec((BO, B), lambda j: (j, 0)),
        out_shape=jax.ShapeDtypeStruct((O, B), jnp.float32),
        scratch_shapes=[pltpu.VMEM((B, E), jnp.bfloat16)],
    )(Wbf, x, b2)


def kernel(label, emb_table, W, b):
    B = label.shape[0]
    E = emb_table.shape[1]
    O = W.shape[0]
    idx2 = jnp.reshape(label.astype(jnp.int32), (_NW, B // _NW))
    Wbf = _cast_bf16(W)
    Wbf, idx2 = jax.lax.optimization_barrier((Wbf, idx2))
    x = _gather_x(emb_table, idx2, B, E)
    outT = _mm_silu_t(Wbf, x, jnp.reshape(b, (O, 1)))
    return jnp.reshape(outT.T, (B, 1, 4, 32, 32))
